# baseline (speedup 1.0000x reference)
import functools

import jax
import jax.numpy as jnp
from jax import lax
from jax.experimental import pallas as pl
from jax.experimental.pallas import tpu as pltpu
from jax.experimental.pallas import tpu_sc as plsc

_NS = 16
_CH = 80


def _sc_hist_body(NP, SL, KCH, dst_hbm, ones_hbm, zeros_hbm, hist_out,
                  dst_v, ones_v, hist_sh):
    s = lax.axis_index("s")
    pltpu.sync_copy(ones_hbm, ones_v)
    pltpu.sync_copy(zeros_hbm.at[pl.ds(s * SL, SL)],
                    hist_sh.at[pl.ds(s * SL, SL)])
    pltpu.sync_copy(dst_hbm.at[s], dst_v)
    plsc.subcore_barrier()

    def body(i, carry):
        pltpu.sync_copy(ones_v, hist_sh.at[dst_v.at[i]], add=True)
        return carry

    lax.fori_loop(0, KCH, body, 0)
    plsc.subcore_barrier()
    pltpu.sync_copy(hist_sh.at[pl.ds(s * SL, SL)],
                    hist_out.at[pl.ds(s * SL, SL)])


def _sc_agg_body(NP, SL, KCH, OUT, y_hbm, src_hbm, dst_hbm, zeros_hbm,
                 agg_out, src_v, dst_v, r0, r1, r2, r3, agg_sh,
                 g0, g1, g2, g3, s0, s1, s2, s3):
    s = lax.axis_index("s")
    pltpu.sync_copy(zeros_hbm.at[pl.ds(s * SL, SL)],
                    agg_sh.at[pl.ds(s * SL, SL)])
    pltpu.sync_copy(src_hbm.at[s], src_v)
    pltpu.sync_copy(dst_hbm.at[s], dst_v)
    plsc.subcore_barrier()

    bufs = (r0, r1, r2, r3)
    gsem = (g0, g1, g2, g3)
    ssem = (s0, s1, s2, s3)

    pltpu.async_copy(y_hbm.at[src_v.at[0]], r0, g0)
    pltpu.async_copy(y_hbm.at[src_v.at[1]], r1, g1)

    def step(i, b, b2):
        pltpu.make_async_copy(y_hbm.at[src_v.at[i]], bufs[b], gsem[b]).wait()
        pltpu.async_copy(bufs[b], agg_sh.at[dst_v.at[i]], ssem[b], add=True)

        @pl.when((i >= 2) & (i + 2 < KCH))
        def _():
            pltpu.make_async_copy(bufs[b2], agg_sh.at[dst_v.at[i - 2]],
                                  ssem[b2]).wait()
            pltpu.async_copy(y_hbm.at[src_v.at[i + 2]], bufs[b2], gsem[b2])

        @pl.when((i < 2) & (i + 2 < KCH))
        def _():
            pltpu.async_copy(y_hbm.at[src_v.at[i + 2]], bufs[b2], gsem[b2])

    def body(i, carry):
        for r in range(4):
            @pl.when(lax.rem(i, 4) == r)
            def _(r=r):
                step(i, r, (r + 2) % 4)
        return carry

    lax.fori_loop(0, KCH, body, 0)
    for j in range(max(0, KCH - 4), KCH):
        pltpu.make_async_copy(bufs[j % 4], agg_sh.at[dst_v.at[j]],
                              ssem[j % 4]).wait()
    plsc.subcore_barrier()
    pltpu.sync_copy(agg_sh.at[pl.ds(s * SL, SL)],
                    agg_out.at[pl.ds(s * SL, SL)])


def _tc_pre_body(D, xp_ref, xn_ref, g_ref, be_ref, W_ref, ha_ref, hb_ref,
                 y_ref):
    g = g_ref[0:1, :]
    be = be_ref[0:1, :]

    def ln(x):
        mu = jnp.mean(x, axis=-1, keepdims=True)
        xc = x - mu
        var = jnp.mean(xc * xc, axis=-1, keepdims=True)
        return xc * lax.rsqrt(var + 1e-5) * g + be

    a = ln(xp_ref[...])
    b2 = ln(xn_ref[...])
    xw = (jnp.dot(a, W_ref[0:D, :], preferred_element_type=jnp.float32)
          + jnp.dot(b2, W_ref[D:2 * D, :], preferred_element_type=jnp.float32))
    deg = 1.0 + ha_ref[:, 0:1] + hb_ref[:, 0:1]
    y_ref[...] = xw * lax.rsqrt(deg)


def _tc_post_body(agga_ref, aggb_ref, y_ref, ha_ref, hb_ref, b_ref, out_ref):
    deg = 1.0 + ha_ref[:, 0:1] + hb_ref[:, 0:1]
    dinv = lax.rsqrt(deg)
    acc = agga_ref[...] + aggb_ref[...] + y_ref[...]
    out_ref[...] = acc * dinv + b_ref[0:1, :]


def kernel(x_prev, x_same, x_next, edge_index, gamma, beta, W, b):
    N, D = x_prev.shape
    OUT = W.shape[1]
    E = edge_index.shape[1]
    EH = E // 2
    KCH = EH // (_NS * _CH)
    assert 2 * _NS * _CH * KCH == E
    NP = ((N + 127) // 128) * 128
    SL = NP // _NS
    RB = 1000
    GRID = N // RB

    f32 = jnp.float32
    src4 = edge_index[0].reshape(2, _NS, KCH, _CH)
    dst4 = edge_index[1].reshape(2, _NS, KCH, _CH)
    ones8 = jnp.ones((_CH, 8), f32)
    zeros8 = jnp.zeros((NP, 8), f32)
    zerosR = jnp.zeros((NP, OUT), f32)
    g2 = jnp.broadcast_to(gamma.reshape(1, D), (8, D))
    be2 = jnp.broadcast_to(beta.reshape(1, D), (8, D))
    b2 = jnp.broadcast_to(b.reshape(1, OUT), (8, OUT))

    mesh = plsc.VectorSubcoreMesh(core_axis_name="c", subcore_axis_name="s",
                                  num_cores=1, num_subcores=_NS)

    hist_call = pl.kernel(
        functools.partial(_sc_hist_body, NP, SL, KCH),
        out_type=jax.ShapeDtypeStruct((NP, 8), f32),
        mesh=mesh,
        compiler_params=pltpu.CompilerParams(use_tc_tiling_on_sc=False),
        scratch_types=[
            pltpu.VMEM((KCH, _CH), jnp.int32),
            pltpu.VMEM((_CH, 8), f32),
            pltpu.VMEM_SHARED((NP, 8), f32),
        ],
    )
    hist_a = hist_call(dst4[0], ones8, zeros8)
    hist_b = hist_call(dst4[1], ones8, zeros8)

    y = pl.pallas_call(
        functools.partial(_tc_pre_body, D),
        grid=(GRID,),
        in_specs=[
            pl.BlockSpec((RB, D), lambda i: (i, 0)),
            pl.BlockSpec((RB, D), lambda i: (i, 0)),
            pl.BlockSpec((8, D), lambda i: (0, 0)),
            pl.BlockSpec((8, D), lambda i: (0, 0)),
            pl.BlockSpec((2 * D, OUT), lambda i: (0, 0)),
            pl.BlockSpec((RB, 8), lambda i: (i, 0)),
            pl.BlockSpec((RB, 8), lambda i: (i, 0)),
        ],
        out_specs=pl.BlockSpec((RB, OUT), lambda i: (i, 0)),
        out_shape=jax.ShapeDtypeStruct((N, OUT), f32),
    )(x_prev, x_next, g2, be2, W, hist_a, hist_b)

    agg_call = pl.kernel(
        functools.partial(_sc_agg_body, NP, SL, KCH, OUT),
        out_type=jax.ShapeDtypeStruct((NP, OUT), f32),
        mesh=mesh,
        compiler_params=pltpu.CompilerParams(use_tc_tiling_on_sc=False),
        scratch_types=[
            pltpu.VMEM((KCH, _CH), jnp.int32),
            pltpu.VMEM((KCH, _CH), jnp.int32),
            pltpu.VMEM((_CH, OUT), f32),
            pltpu.VMEM((_CH, OUT), f32),
            pltpu.VMEM((_CH, OUT), f32),
            pltpu.VMEM((_CH, OUT), f32),
            pltpu.VMEM_SHARED((NP, OUT), f32),
        ] + [pltpu.SemaphoreType.DMA] * 8,
    )
    agg_a = agg_call(y, src4[0], dst4[0], zerosR)
    agg_b = agg_call(y, src4[1], dst4[1], zerosR)

    out = pl.pallas_call(
        _tc_post_body,
        grid=(GRID,),
        in_specs=[
            pl.BlockSpec((RB, OUT), lambda i: (i, 0)),
            pl.BlockSpec((RB, OUT), lambda i: (i, 0)),
            pl.BlockSpec((RB, OUT), lambda i: (i, 0)),
            pl.BlockSpec((RB, 8), lambda i: (i, 0)),
            pl.BlockSpec((RB, 8), lambda i: (i, 0)),
            pl.BlockSpec((8, OUT), lambda i: (0, 0)),
        ],
        out_specs=pl.BlockSpec((RB, OUT), lambda i: (i, 0)),
        out_shape=jax.ShapeDtypeStruct((N, OUT), f32),
    )(agg_a, agg_b, y, hist_a, hist_b, b2)

    return out


# speedup vs baseline: 1.5181x; 1.5181x over previous
import functools

import jax
import jax.numpy as jnp
from jax import lax
from jax.experimental import pallas as pl
from jax.experimental.pallas import tpu as pltpu
from jax.experimental.pallas import tpu_sc as plsc

_NC = 2
_NS = 16
_NW = _NC * _NS
_CH = 128
_CT = 16


def _sc_hist_body(NP, SL, KCH, dst_hbm, dstt_hbm, ones_hbm, zeros_hbm,
                  hist_out, dst_v, dstt_v, ones_v, hist_sh):
    c = lax.axis_index("c")
    s = lax.axis_index("s")
    w = s * _NC + c
    pltpu.sync_copy(ones_hbm, ones_v)
    pltpu.sync_copy(zeros_hbm.at[pl.ds(s * SL, SL)],
                    hist_sh.at[pl.ds(s * SL, SL)])
    pltpu.sync_copy(dst_hbm.at[w], dst_v)
    pltpu.sync_copy(dstt_hbm.at[w], dstt_v)
    plsc.subcore_barrier()

    def body(i, carry):
        pltpu.sync_copy(ones_v, hist_sh.at[dst_v.at[i]], add=True)
        return carry

    lax.fori_loop(0, KCH, body, 0)
    pltpu.sync_copy(ones_v.at[pl.ds(0, _CT)], hist_sh.at[dstt_v], add=True)
    plsc.subcore_barrier()
    pltpu.sync_copy(hist_sh.at[pl.ds(s * SL, SL)],
                    hist_out.at[c, pl.ds(s * SL, SL)])


def _sc_agg_body(NP, SL, KCH, OUT, y_hbm, src_hbm, dst_hbm, srct_hbm,
                 dstt_hbm, zeros_hbm, agg_out, src_v, dst_v, srct_v, dstt_v,
                 r0, r1, r2, r3, rt, agg_sh,
                 g0, g1, g2, g3, s0, s1, s2, s3, gt):
    c = lax.axis_index("c")
    s = lax.axis_index("s")
    w = s * _NC + c
    pltpu.sync_copy(zeros_hbm.at[pl.ds(s * SL, SL)],
                    agg_sh.at[pl.ds(s * SL, SL)])
    pltpu.sync_copy(src_hbm.at[w], src_v)
    pltpu.sync_copy(dst_hbm.at[w], dst_v)
    pltpu.sync_copy(srct_hbm.at[w], srct_v)
    pltpu.sync_copy(dstt_hbm.at[w], dstt_v)
    plsc.subcore_barrier()

    bufs = (r0, r1, r2, r3)
    gsem = (g0, g1, g2, g3)
    ssem = (s0, s1, s2, s3)

    pltpu.async_copy(y_hbm.at[src_v.at[0]], r0, g0)
    pltpu.async_copy(y_hbm.at[src_v.at[1]], r1, g1)

    def step(i, b, b2):
        pltpu.make_async_copy(y_hbm.at[src_v.at[i]], bufs[b], gsem[b]).wait()
        pltpu.async_copy(bufs[b], agg_sh.at[dst_v.at[i]], ssem[b], add=True)

        @pl.when((i >= 2) & (i + 2 < KCH))
        def _():
            pltpu.make_async_copy(bufs[b2], agg_sh.at[dst_v.at[i - 2]],
                                  ssem[b2]).wait()
            pltpu.async_copy(y_hbm.at[src_v.at[i + 2]], bufs[b2], gsem[b2])

        @pl.when((i < 2) & (i + 2 < KCH))
        def _():
            pltpu.async_copy(y_hbm.at[src_v.at[i + 2]], bufs[b2], gsem[b2])

    def body(i, carry):
        for r in range(4):
            @pl.when(lax.rem(i, 4) == r)
            def _(r=r):
                step(i, r, (r + 2) % 4)
        return carry

    pltpu.async_copy(y_hbm.at[srct_v], rt, gt)
    lax.fori_loop(0, KCH, body, 0)
    for j in range(max(0, KCH - 4), KCH):
        pltpu.make_async_copy(bufs[j % 4], agg_sh.at[dst_v.at[j]],
                              ssem[j % 4]).wait()
    pltpu.make_async_copy(y_hbm.at[srct_v], rt, gt).wait()
    pltpu.sync_copy(rt, agg_sh.at[dstt_v], add=True)
    plsc.subcore_barrier()
    pltpu.sync_copy(agg_sh.at[pl.ds(s * SL, SL)],
                    agg_out.at[c, pl.ds(s * SL, SL)])


def _tc_pre_body(D, xp_ref, xn_ref, g_ref, be_ref, W_ref, hist_ref, y_ref):
    g = g_ref[0:1, :]
    be = be_ref[0:1, :]

    def ln(x):
        mu = jnp.mean(x, axis=-1, keepdims=True)
        xc = x - mu
        var = jnp.mean(xc * xc, axis=-1, keepdims=True)
        return xc * lax.rsqrt(var + 1e-5) * g + be

    a = ln(xp_ref[...])
    b2 = ln(xn_ref[...])
    xw = (jnp.dot(a, W_ref[0:D, :], preferred_element_type=jnp.float32)
          + jnp.dot(b2, W_ref[D:2 * D, :], preferred_element_type=jnp.float32))
    h = hist_ref[...]
    deg = 1.0 + h[0, :, 0:1] + h[1, :, 0:1]
    y_ref[...] = xw * lax.rsqrt(deg)


def _tc_post_body(agg_ref, y_ref, hist_ref, b_ref, out_ref):
    h = hist_ref[...]
    deg = 1.0 + h[0, :, 0:1] + h[1, :, 0:1]
    dinv = lax.rsqrt(deg)
    acc = agg_ref[0] + agg_ref[1] + y_ref[...]
    out_ref[...] = acc * dinv + b_ref[0:1, :]


def kernel(x_prev, x_same, x_next, edge_index, gamma, beta, W, b):
    N, D = x_prev.shape
    OUT = W.shape[1]
    E = edge_index.shape[1]
    EW = E // _NW
    KCH = (EW - _CT) // _CH
    assert _NW * (KCH * _CH + _CT) == E
    NP = ((N + 127) // 128) * 128
    SL = NP // _NS
    RB = 1000
    GRID = N // RB

    f32 = jnp.float32
    src_w = edge_index[0].reshape(_NW, EW)
    dst_w = edge_index[1].reshape(_NW, EW)
    src3 = src_w[:, :KCH * _CH].reshape(_NW, KCH, _CH)
    dst3 = dst_w[:, :KCH * _CH].reshape(_NW, KCH, _CH)
    srct = src_w[:, KCH * _CH:]
    dstt = dst_w[:, KCH * _CH:]
    ones8 = jnp.ones((_CH, 8), f32)
    zeros8 = jnp.zeros((NP, 8), f32)
    zerosR = jnp.zeros((NP, OUT), f32)
    g2 = jnp.broadcast_to(gamma.reshape(1, D), (8, D))
    be2 = jnp.broadcast_to(beta.reshape(1, D), (8, D))
    b2 = jnp.broadcast_to(b.reshape(1, OUT), (8, OUT))

    mesh = plsc.VectorSubcoreMesh(core_axis_name="c", subcore_axis_name="s",
                                  num_cores=_NC, num_subcores=_NS)

    hist = pl.kernel(
        functools.partial(_sc_hist_body, NP, SL, KCH),
        out_type=jax.ShapeDtypeStruct((_NC, NP, 8), f32),
        mesh=mesh,
        compiler_params=pltpu.CompilerParams(use_tc_tiling_on_sc=False),
        scratch_types=[
            pltpu.VMEM((KCH, _CH), jnp.int32),
            pltpu.VMEM((_CT,), jnp.int32),
            pltpu.VMEM((_CH, 8), f32),
            pltpu.VMEM_SHARED((NP, 8), f32),
        ],
    )(dst3, dstt, ones8, zeros8)

    y = pl.pallas_call(
        functools.partial(_tc_pre_body, D),
        grid=(GRID,),
        in_specs=[
            pl.BlockSpec((RB, D), lambda i: (i, 0)),
            pl.BlockSpec((RB, D), lambda i: (i, 0)),
            pl.BlockSpec((8, D), lambda i: (0, 0)),
            pl.BlockSpec((8, D), lambda i: (0, 0)),
            pl.BlockSpec((2 * D, OUT), lambda i: (0, 0)),
            pl.BlockSpec((_NC, RB, 8), lambda i: (0, i, 0)),
        ],
        out_specs=pl.BlockSpec((RB, OUT), lambda i: (i, 0)),
        out_shape=jax.ShapeDtypeStruct((N, OUT), f32),
    )(x_prev, x_next, g2, be2, W, hist)

    agg = pl.kernel(
        functools.partial(_sc_agg_body, NP, SL, KCH, OUT),
        out_type=jax.ShapeDtypeStruct((_NC, NP, OUT), f32),
        mesh=mesh,
        compiler_params=pltpu.CompilerParams(use_tc_tiling_on_sc=False),
        scratch_types=[
            pltpu.VMEM((KCH, _CH), jnp.int32),
            pltpu.VMEM((KCH, _CH), jnp.int32),
            pltpu.VMEM((_CT,), jnp.int32),
            pltpu.VMEM((_CT,), jnp.int32),
            pltpu.VMEM((_CH, OUT), f32),
            pltpu.VMEM((_CH, OUT), f32),
            pltpu.VMEM((_CH, OUT), f32),
            pltpu.VMEM((_CH, OUT), f32),
            pltpu.VMEM((_CT, OUT), f32),
            pltpu.VMEM_SHARED((NP, OUT), f32),
        ] + [pltpu.SemaphoreType.DMA] * 9,
    )(y, src3, dst3, srct, dstt, zerosR)

    out = pl.pallas_call(
        _tc_post_body,
        grid=(GRID,),
        in_specs=[
            pl.BlockSpec((_NC, RB, OUT), lambda i: (0, i, 0)),
            pl.BlockSpec((RB, OUT), lambda i: (i, 0)),
            pl.BlockSpec((_NC, RB, 8), lambda i: (0, i, 0)),
            pl.BlockSpec((8, OUT), lambda i: (0, 0)),
        ],
        out_specs=pl.BlockSpec((RB, OUT), lambda i: (i, 0)),
        out_shape=jax.ShapeDtypeStruct((N, OUT), f32),
    )(agg, y, hist, b2)

    return out


# speedup vs baseline: 1.5630x; 1.0296x over previous
import functools

import jax
import jax.numpy as jnp
from jax import lax
from jax.experimental import pallas as pl
from jax.experimental.pallas import tpu as pltpu
from jax.experimental.pallas import tpu_sc as plsc

_NC = 2
_NS = 16
_NW = _NC * _NS
_CH = 128
_CT = 16
_RING = 6


def _sc_hist_body(NP, SL, KCH, dst_hbm, dstt_hbm, ones_hbm, zeros_hbm,
                  hist_out, dst_v, dstt_v, ones_v, hist_sh):
    c = lax.axis_index("c")
    s = lax.axis_index("s")
    w = s * _NC + c
    pltpu.sync_copy(ones_hbm, ones_v)
    pltpu.sync_copy(zeros_hbm.at[pl.ds(s * SL, SL)],
                    hist_sh.at[pl.ds(s * SL, SL)])
    pltpu.sync_copy(dst_hbm.at[w], dst_v)
    pltpu.sync_copy(dstt_hbm.at[w], dstt_v)
    plsc.subcore_barrier()

    def body(i, carry):
        pltpu.sync_copy(ones_v, hist_sh.at[dst_v.at[i]], add=True)
        return carry

    lax.fori_loop(0, KCH, body, 0)
    pltpu.sync_copy(ones_v.at[pl.ds(0, _CT)], hist_sh.at[dstt_v], add=True)
    plsc.subcore_barrier()
    pltpu.sync_copy(hist_sh.at[pl.ds(s * SL, SL)],
                    hist_out.at[c, pl.ds(s * SL, SL)])


def _sc_agg_body(NP, SL, KCH, OUT, y_hbm, src_hbm, dst_hbm, srct_hbm,
                 dstt_hbm, zeros_hbm, agg_out, src_v, dst_v, srct_v, dstt_v,
                 r0, r1, r2, r3, r4, r5, rt, agg_sh,
                 g0, g1, g2, g3, g4, g5, s0, s1, s2, s3, s4, s5, gt):
    c = lax.axis_index("c")
    s = lax.axis_index("s")
    w = s * _NC + c
    pltpu.sync_copy(zeros_hbm.at[pl.ds(s * SL, SL)],
                    agg_sh.at[pl.ds(s * SL, SL)])
    pltpu.sync_copy(src_hbm.at[w], src_v)
    pltpu.sync_copy(dst_hbm.at[w], dst_v)
    pltpu.sync_copy(srct_hbm.at[w], srct_v)
    pltpu.sync_copy(dstt_hbm.at[w], dstt_v)
    plsc.subcore_barrier()

    bufs = (r0, r1, r2, r3, r4, r5)
    gsem = (g0, g1, g2, g3, g4, g5)
    ssem = (s0, s1, s2, s3, s4, s5)
    H = _RING // 2

    for j in range(H):
        pltpu.async_copy(y_hbm.at[src_v.at[j]], bufs[j], gsem[j])

    def step(i, b, b2):
        pltpu.make_async_copy(y_hbm.at[src_v.at[i]], bufs[b], gsem[b]).wait()
        pltpu.async_copy(bufs[b], agg_sh.at[dst_v.at[i]], ssem[b], add=True)

        @pl.when((i >= H) & (i + H < KCH))
        def _():
            pltpu.make_async_copy(bufs[b2], agg_sh.at[dst_v.at[i - H]],
                                  ssem[b2]).wait()
            pltpu.async_copy(y_hbm.at[src_v.at[i + H]], bufs[b2], gsem[b2])

        @pl.when((i < H) & (i + H < KCH))
        def _():
            pltpu.async_copy(y_hbm.at[src_v.at[i + H]], bufs[b2], gsem[b2])

    def body(i, carry):
        for r in range(_RING):
            @pl.when(lax.rem(i, _RING) == r)
            def _(r=r):
                step(i, r, (r + H) % _RING)
        return carry

    pltpu.async_copy(y_hbm.at[srct_v], rt, gt)
    lax.fori_loop(0, KCH, body, 0)
    for j in range(max(0, KCH - _RING), KCH):
        pltpu.make_async_copy(bufs[j % _RING], agg_sh.at[dst_v.at[j]],
                              ssem[j % _RING]).wait()
    pltpu.make_async_copy(y_hbm.at[srct_v], rt, gt).wait()
    pltpu.sync_copy(rt, agg_sh.at[dstt_v], add=True)
    plsc.subcore_barrier()
    pltpu.sync_copy(agg_sh.at[pl.ds(s * SL, SL)],
                    agg_out.at[c, pl.ds(s * SL, SL)])


def _tc_pre_body(D, xp_ref, xn_ref, g_ref, be_ref, W_ref, hist_ref, y_ref):
    g = g_ref[0:1, :]
    be = be_ref[0:1, :]

    def ln(x):
        mu = jnp.mean(x, axis=-1, keepdims=True)
        xc = x - mu
        var = jnp.mean(xc * xc, axis=-1, keepdims=True)
        return xc * lax.rsqrt(var + 1e-5) * g + be

    a = ln(xp_ref[...])
    b2 = ln(xn_ref[...])
    xw = (jnp.dot(a, W_ref[0:D, :], preferred_element_type=jnp.float32)
          + jnp.dot(b2, W_ref[D:2 * D, :], preferred_element_type=jnp.float32))
    h = hist_ref[...]
    deg = 1.0 + h[0, :, 0:1] + h[1, :, 0:1]
    y_ref[...] = xw * lax.rsqrt(deg)


def _tc_post_body(agg_ref, y_ref, hist_ref, b_ref, out_ref):
    h = hist_ref[...]
    deg = 1.0 + h[0, :, 0:1] + h[1, :, 0:1]
    dinv = lax.rsqrt(deg)
    acc = agg_ref[0] + agg_ref[1] + y_ref[...]
    out_ref[...] = acc * dinv + b_ref[0:1, :]


def kernel(x_prev, x_same, x_next, edge_index, gamma, beta, W, b):
    N, D = x_prev.shape
    OUT = W.shape[1]
    E = edge_index.shape[1]
    EW = E // _NW
    KCH = (EW - _CT) // _CH
    assert _NW * (KCH * _CH + _CT) == E
    NP = ((N + 127) // 128) * 128
    SL = NP // _NS
    RB = 1000
    GRID = N // RB

    f32 = jnp.float32
    src_w = edge_index[0].reshape(_NW, EW)
    dst_w = edge_index[1].reshape(_NW, EW)
    src3 = src_w[:, :KCH * _CH].reshape(_NW, KCH, _CH)
    dst3 = dst_w[:, :KCH * _CH].reshape(_NW, KCH, _CH)
    srct = src_w[:, KCH * _CH:]
    dstt = dst_w[:, KCH * _CH:]
    ones8 = jnp.ones((_CH, 8), f32)
    zeros8 = jnp.zeros((NP, 8), f32)
    zerosR = jnp.zeros((NP, OUT), f32)
    g2 = jnp.broadcast_to(gamma.reshape(1, D), (8, D))
    be2 = jnp.broadcast_to(beta.reshape(1, D), (8, D))
    b2 = jnp.broadcast_to(b.reshape(1, OUT), (8, OUT))

    mesh = plsc.VectorSubcoreMesh(core_axis_name="c", subcore_axis_name="s",
                                  num_cores=_NC, num_subcores=_NS)

    hist = pl.kernel(
        functools.partial(_sc_hist_body, NP, SL, KCH),
        out_type=jax.ShapeDtypeStruct((_NC, NP, 8), f32),
        mesh=mesh,
        compiler_params=pltpu.CompilerParams(use_tc_tiling_on_sc=False),
        scratch_types=[
            pltpu.VMEM((KCH, _CH), jnp.int32),
            pltpu.VMEM((_CT,), jnp.int32),
            pltpu.VMEM((_CH, 8), f32),
            pltpu.VMEM_SHARED((NP, 8), f32),
        ],
    )(dst3, dstt, ones8, zeros8)

    y = pl.pallas_call(
        functools.partial(_tc_pre_body, D),
        grid=(GRID,),
        in_specs=[
            pl.BlockSpec((RB, D), lambda i: (i, 0)),
            pl.BlockSpec((RB, D), lambda i: (i, 0)),
            pl.BlockSpec((8, D), lambda i: (0, 0)),
            pl.BlockSpec((8, D), lambda i: (0, 0)),
            pl.BlockSpec((2 * D, OUT), lambda i: (0, 0)),
            pl.BlockSpec((_NC, RB, 8), lambda i: (0, i, 0)),
        ],
        out_specs=pl.BlockSpec((RB, OUT), lambda i: (i, 0)),
        out_shape=jax.ShapeDtypeStruct((N, OUT), f32),
    )(x_prev, x_next, g2, be2, W, hist)

    agg = pl.kernel(
        functools.partial(_sc_agg_body, NP, SL, KCH, OUT),
        out_type=jax.ShapeDtypeStruct((_NC, NP, OUT), f32),
        mesh=mesh,
        compiler_params=pltpu.CompilerParams(use_tc_tiling_on_sc=False),
        scratch_types=[
            pltpu.VMEM((KCH, _CH), jnp.int32),
            pltpu.VMEM((KCH, _CH), jnp.int32),
            pltpu.VMEM((_CT,), jnp.int32),
            pltpu.VMEM((_CT,), jnp.int32),
        ] + [pltpu.VMEM((_CH, OUT), f32)] * _RING + [
            pltpu.VMEM((_CT, OUT), f32),
            pltpu.VMEM_SHARED((NP, OUT), f32),
        ] + [pltpu.SemaphoreType.DMA] * (2 * _RING + 1),
    )(y, src3, dst3, srct, dstt, zerosR)

    out = pl.pallas_call(
        _tc_post_body,
        grid=(GRID,),
        in_specs=[
            pl.BlockSpec((_NC, RB, OUT), lambda i: (0, i, 0)),
            pl.BlockSpec((RB, OUT), lambda i: (i, 0)),
            pl.BlockSpec((_NC, RB, 8), lambda i: (0, i, 0)),
            pl.BlockSpec((8, OUT), lambda i: (0, 0)),
        ],
        out_specs=pl.BlockSpec((RB, OUT), lambda i: (i, 0)),
        out_shape=jax.ShapeDtypeStruct((N, OUT), f32),
    )(agg, y, hist, b2)

    return out


# speedup vs baseline: 1.5911x; 1.0179x over previous
import functools

import jax
import jax.numpy as jnp
from jax import lax
from jax.experimental import pallas as pl
from jax.experimental.pallas import tpu as pltpu
from jax.experimental.pallas import tpu_sc as plsc

_NC = 2
_NS = 16
_NW = _NC * _NS
_CH = 128
_CT = 16
_RING = 6


def _sc_hist_body(NP, SL, KCH, dst_hbm, dstt_hbm, ones_hbm, zeros_hbm,
                  hist_out, dst_v, dstt_v, ones_v, hist_sh, hq):
    c = lax.axis_index("c")
    s = lax.axis_index("s")
    w = s * _NC + c
    pltpu.sync_copy(ones_hbm, ones_v)
    pltpu.sync_copy(zeros_hbm.at[pl.ds(s * SL, SL)],
                    hist_sh.at[pl.ds(s * SL, SL)])
    pltpu.sync_copy(dst_hbm.at[w], dst_v)
    pltpu.sync_copy(dstt_hbm.at[w], dstt_v)
    plsc.subcore_barrier()

    def body(i, carry):
        pltpu.async_copy(ones_v, hist_sh.at[dst_v.at[i]], hq, add=True)

        @pl.when(i >= 4)
        def _():
            pltpu.make_async_copy(ones_v, hist_sh.at[dst_v.at[i]], hq).wait()

        return carry

    lax.fori_loop(0, KCH, body, 0)
    for _ in range(min(4, KCH)):
        pltpu.make_async_copy(ones_v, hist_sh.at[dst_v.at[0]], hq).wait()
    pltpu.sync_copy(ones_v.at[pl.ds(0, _CT)], hist_sh.at[dstt_v], add=True)
    plsc.subcore_barrier()
    pltpu.sync_copy(hist_sh.at[pl.ds(s * SL, SL)],
                    hist_out.at[c, pl.ds(s * SL, SL)])


def _sc_agg_body(NP, SL, KCH, OUT, y_hbm, src_hbm, dst_hbm, srct_hbm,
                 dstt_hbm, zeros_hbm, agg_out, src_v, dst_v, srct_v, dstt_v,
                 r0, r1, r2, r3, r4, r5, rt, agg_sh,
                 g0, g1, g2, g3, g4, g5, s0, s1, s2, s3, s4, s5, gt):
    c = lax.axis_index("c")
    s = lax.axis_index("s")
    w = s * _NC + c
    pltpu.sync_copy(zeros_hbm.at[pl.ds(s * SL, SL)],
                    agg_sh.at[pl.ds(s * SL, SL)])
    pltpu.sync_copy(src_hbm.at[w], src_v)
    pltpu.sync_copy(dst_hbm.at[w], dst_v)
    pltpu.sync_copy(srct_hbm.at[w], srct_v)
    pltpu.sync_copy(dstt_hbm.at[w], dstt_v)
    plsc.subcore_barrier()

    bufs = (r0, r1, r2, r3, r4, r5)
    gsem = (g0, g1, g2, g3, g4, g5)
    ssem = (s0, s1, s2, s3, s4, s5)
    H = _RING // 2

    for j in range(H):
        pltpu.async_copy(y_hbm.at[src_v.at[j]], bufs[j], gsem[j])

    def step(i, b, b2):
        pltpu.make_async_copy(y_hbm.at[src_v.at[i]], bufs[b], gsem[b]).wait()
        pltpu.async_copy(bufs[b], agg_sh.at[dst_v.at[i]], ssem[b], add=True)

        @pl.when((i >= H) & (i + H < KCH))
        def _():
            pltpu.make_async_copy(bufs[b2], agg_sh.at[dst_v.at[i - H]],
                                  ssem[b2]).wait()
            pltpu.async_copy(y_hbm.at[src_v.at[i + H]], bufs[b2], gsem[b2])

        @pl.when((i < H) & (i + H < KCH))
        def _():
            pltpu.async_copy(y_hbm.at[src_v.at[i + H]], bufs[b2], gsem[b2])

    def body(i, carry):
        for r in range(_RING):
            @pl.when(lax.rem(i, _RING) == r)
            def _(r=r):
                step(i, r, (r + H) % _RING)
        return carry

    pltpu.async_copy(y_hbm.at[srct_v], rt, gt)
    lax.fori_loop(0, KCH, body, 0)
    for j in range(max(0, KCH - _RING), KCH):
        pltpu.make_async_copy(bufs[j % _RING], agg_sh.at[dst_v.at[j]],
                              ssem[j % _RING]).wait()
    pltpu.make_async_copy(y_hbm.at[srct_v], rt, gt).wait()
    pltpu.sync_copy(rt, agg_sh.at[dstt_v], add=True)
    plsc.subcore_barrier()
    pltpu.sync_copy(agg_sh.at[pl.ds(s * SL, SL)],
                    agg_out.at[c, pl.ds(s * SL, SL)])


def _tc_pre_body(D, xp_ref, xn_ref, g_ref, be_ref, W_ref, hist_ref, y_ref):
    g = g_ref[0:1, :]
    be = be_ref[0:1, :]

    def ln(x):
        mu = jnp.mean(x, axis=-1, keepdims=True)
        xc = x - mu
        var = jnp.mean(xc * xc, axis=-1, keepdims=True)
        return xc * lax.rsqrt(var + 1e-5) * g + be

    a = ln(xp_ref[...])
    b2 = ln(xn_ref[...])
    xw = (jnp.dot(a, W_ref[0:D, :], preferred_element_type=jnp.float32)
          + jnp.dot(b2, W_ref[D:2 * D, :], preferred_element_type=jnp.float32))
    h = hist_ref[...]
    deg = 1.0 + h[0, :, 0:1] + h[1, :, 0:1]
    y_ref[...] = xw * lax.rsqrt(deg)


def _tc_post_body(agg_ref, y_ref, hist_ref, b_ref, out_ref):
    h = hist_ref[...]
    deg = 1.0 + h[0, :, 0:1] + h[1, :, 0:1]
    dinv = lax.rsqrt(deg)
    acc = agg_ref[0] + agg_ref[1] + y_ref[...]
    out_ref[...] = acc * dinv + b_ref[0:1, :]


def kernel(x_prev, x_same, x_next, edge_index, gamma, beta, W, b):
    N, D = x_prev.shape
    OUT = W.shape[1]
    E = edge_index.shape[1]
    EW = E // _NW
    KCH = (EW - _CT) // _CH
    assert _NW * (KCH * _CH + _CT) == E
    NP = ((N + 127) // 128) * 128
    SL = NP // _NS
    RB = 1000
    GRID = N // RB

    f32 = jnp.float32
    src_w = edge_index[0].reshape(_NW, EW)
    dst_w = edge_index[1].reshape(_NW, EW)
    src3 = src_w[:, :KCH * _CH].reshape(_NW, KCH, _CH)
    dst3 = dst_w[:, :KCH * _CH].reshape(_NW, KCH, _CH)
    srct = src_w[:, KCH * _CH:]
    dstt = dst_w[:, KCH * _CH:]
    ones8 = jnp.ones((_CH, 8), f32)
    zeros8 = jnp.zeros((NP, 8), f32)
    zerosR = jnp.zeros((NP, OUT), f32)
    g2 = jnp.broadcast_to(gamma.reshape(1, D), (8, D))
    be2 = jnp.broadcast_to(beta.reshape(1, D), (8, D))
    b2 = jnp.broadcast_to(b.reshape(1, OUT), (8, OUT))

    mesh = plsc.VectorSubcoreMesh(core_axis_name="c", subcore_axis_name="s",
                                  num_cores=_NC, num_subcores=_NS)

    hist = pl.kernel(
        functools.partial(_sc_hist_body, NP, SL, KCH),
        out_type=jax.ShapeDtypeStruct((_NC, NP, 8), f32),
        mesh=mesh,
        compiler_params=pltpu.CompilerParams(use_tc_tiling_on_sc=False),
        scratch_types=[
            pltpu.VMEM((KCH, _CH), jnp.int32),
            pltpu.VMEM((_CT,), jnp.int32),
            pltpu.VMEM((_CH, 8), f32),
            pltpu.VMEM_SHARED((NP, 8), f32),
            pltpu.SemaphoreType.DMA,
        ],
    )(dst3, dstt, ones8, zeros8)

    y = pl.pallas_call(
        functools.partial(_tc_pre_body, D),
        grid=(GRID,),
        in_specs=[
            pl.BlockSpec((RB, D), lambda i: (i, 0)),
            pl.BlockSpec((RB, D), lambda i: (i, 0)),
            pl.BlockSpec((8, D), lambda i: (0, 0)),
            pl.BlockSpec((8, D), lambda i: (0, 0)),
            pl.BlockSpec((2 * D, OUT), lambda i: (0, 0)),
            pl.BlockSpec((_NC, RB, 8), lambda i: (0, i, 0)),
        ],
        out_specs=pl.BlockSpec((RB, OUT), lambda i: (i, 0)),
        out_shape=jax.ShapeDtypeStruct((N, OUT), f32),
    )(x_prev, x_next, g2, be2, W, hist)

    agg = pl.kernel(
        functools.partial(_sc_agg_body, NP, SL, KCH, OUT),
        out_type=jax.ShapeDtypeStruct((_NC, NP, OUT), f32),
        mesh=mesh,
        compiler_params=pltpu.CompilerParams(use_tc_tiling_on_sc=False),
        scratch_types=[
            pltpu.VMEM((KCH, _CH), jnp.int32),
            pltpu.VMEM((KCH, _CH), jnp.int32),
            pltpu.VMEM((_CT,), jnp.int32),
            pltpu.VMEM((_CT,), jnp.int32),
        ] + [pltpu.VMEM((_CH, OUT), f32)] * _RING + [
            pltpu.VMEM((_CT, OUT), f32),
            pltpu.VMEM_SHARED((NP, OUT), f32),
        ] + [pltpu.SemaphoreType.DMA] * (2 * _RING + 1),
    )(y, src3, dst3, srct, dstt, zerosR)

    out = pl.pallas_call(
        _tc_post_body,
        grid=(GRID,),
        in_specs=[
            pl.BlockSpec((_NC, RB, OUT), lambda i: (0, i, 0)),
            pl.BlockSpec((RB, OUT), lambda i: (i, 0)),
            pl.BlockSpec((_NC, RB, 8), lambda i: (0, i, 0)),
            pl.BlockSpec((8, OUT), lambda i: (0, 0)),
        ],
        out_specs=pl.BlockSpec((RB, OUT), lambda i: (i, 0)),
        out_shape=jax.ShapeDtypeStruct((N, OUT), f32),
    )(agg, y, hist, b2)

    return out
